# baseline (speedup 1.0000x reference)
import jax
import jax.numpy as jnp
from jax import lax
from jax.experimental import pallas as pl
from jax.experimental.pallas import tpu as pltpu
from jax.experimental.pallas import tpu_sc as plsc

_N = 10000
_E = 160000
_IN = 256
_HID = 512
_OUT = 256
_F = 128
_NC, _NS = 2, 16
_CH = 128
_NP = 10240
_RPS = _NP // _NS
_ZR = 128
_EPS = 10240
_NJ = _EPS // _CH
_NH = 2
_HNJ = _NJ // _NH
_EPAD = _EPS * _NS
_NCHUNK = _EPAD // _CH
_NB = 2
_BM = 400


def _make_sc_agg(with_deg):
  mesh = plsc.VectorSubcoreMesh(core_axis_name="c", subcore_axis_name="s")
  out_type = [jax.ShapeDtypeStruct((_NC, _NP, _F), jnp.float32)]
  scratch = [
      pltpu.VMEM_SHARED((_NP, _F), jnp.float32),
      pltpu.VMEM((_ZR, _F), jnp.float32),
      pltpu.VMEM((_CH,), jnp.int32),
      pltpu.VMEM((_CH,), jnp.int32),
      pltpu.VMEM((_CH, _F), jnp.float32),
      pltpu.SemaphoreType.DMA,
  ]
  if with_deg:
    out_type.append(jax.ShapeDtypeStruct((_NC, _NP), jnp.float32))
    scratch += [
        pltpu.VMEM_SHARED((_NP,), jnp.float32),
        pltpu.VMEM((_RPS,), jnp.float32),
        pltpu.VMEM((_CH,), jnp.float32),
    ]

  def body(rows2n, srcs2, dst1, *rest):
    if with_deg:
      out, deg_out = rest[0], rest[1]
      acc_sh, zbuf, idxs_v, idxd_v, rows_v, sem, deg_sh, zdeg, ones_v = rest[2:]
    else:
      out = rest[0]
      acc_sh, zbuf, idxs_v, idxd_v, rows_v, sem = rest[1:]
    c = lax.axis_index("c")
    s = lax.axis_index("s")
    zero16 = jnp.zeros((16,), jnp.float32)

    def zb(i, carry):
      for j in range(_F // 16):
        zbuf[i, pl.ds(j * 16, 16)] = zero16
      return carry
    lax.fori_loop(0, _ZR, zb, 0)

    r0 = s * _RPS
    for k in range(_RPS // _ZR):
      pltpu.sync_copy(zbuf, acc_sh.at[pl.ds(r0 + k * _ZR, _ZR)])

    if with_deg:
      def zd(i, carry):
        zdeg[pl.ds(i * 16, 16)] = zero16
        return carry
      lax.fori_loop(0, _RPS // 16, zd, 0)
      pltpu.sync_copy(zdeg, deg_sh.at[pl.ds(r0, _RPS)])
      one16 = jnp.full((16,), 1.0, jnp.float32)
      for j in range(_CH // 16):
        ones_v[pl.ds(j * 16, 16)] = one16

    plsc.subcore_barrier()

    def chunk(i, carry):
      k = s + i * _NS
      e0 = k * _CH
      pltpu.sync_copy(srcs2.at[c, pl.ds(e0, _CH)], idxs_v)
      pltpu.sync_copy(dst1.at[pl.ds(e0, _CH)], idxd_v)
      pltpu.async_copy(rows2n.at[idxs_v], rows_v, sem).wait()
      pltpu.sync_copy(rows_v, acc_sh.at[idxd_v], add=True)
      if with_deg:
        @pl.when((k < _NCHUNK // 2) == (c == 0))
        def _deg():
          pltpu.sync_copy(ones_v, deg_sh.at[idxd_v], add=True)
      return carry
    lax.fori_loop(0, _NJ, chunk, 0)

    plsc.subcore_barrier()

    for k in range(_RPS // _ZR):
      rr = r0 + k * _ZR
      pltpu.sync_copy(acc_sh.at[pl.ds(rr, _ZR)], zbuf)
      pltpu.sync_copy(zbuf, out.at[c, pl.ds(rr, _ZR)])
    if with_deg:
      pltpu.sync_copy(deg_sh.at[pl.ds(r0, _RPS)], zdeg)
      pltpu.sync_copy(zdeg, deg_out.at[c, pl.ds(r0, _RPS)])

  return pl.kernel(body, out_type=tuple(out_type), mesh=mesh,
                   scratch_types=tuple(scratch))


_sc_agg_deg = _make_sc_agg(True)
_sc_agg = _make_sc_agg(False)


def _l0_body(x_ref, agg_ref, deg_ref, ws_ref, wn_ref, b_ref, o_ref):
  inv = 1.0 / jnp.maximum(deg_ref[0] + deg_ref[1], 1.0)
  acc = jnp.dot(x_ref[...], ws_ref[...], preferred_element_type=jnp.float32)
  acc += jnp.dot(agg_ref[0] * inv, wn_ref[0:_F, :],
                 preferred_element_type=jnp.float32)
  acc += jnp.dot(agg_ref[1] * inv, wn_ref[_F:2 * _F, :],
                 preferred_element_type=jnp.float32)
  o_ref[...] = jnp.maximum(acc + b_ref[...], 0.0)


def _l1a_body(h_ref, ws_ref, wn_ref, b_ref, part_ref, p_ref):
  h = h_ref[...]
  part_ref[...] = (
      jnp.dot(h, ws_ref[...], preferred_element_type=jnp.float32) + b_ref[...])
  p = jnp.dot(h, wn_ref[...], preferred_element_type=jnp.float32)
  p_ref[:, 0, :] = p[:, 0:_F]
  p_ref[:, 1, :] = p[:, _F:2 * _F]


def _l1b_body(part_ref, agg_ref, deg_ref, o_ref):
  inv = 1.0 / jnp.maximum(deg_ref[0] + deg_ref[1], 1.0)
  o_ref[:, 0:_F] = part_ref[:, 0:_F] + agg_ref[0] * inv
  o_ref[:, _F:2 * _F] = part_ref[:, _F:2 * _F] + agg_ref[1] * inv


def _l0(x, agg0, deg, ws, wn, b):
  return pl.pallas_call(
      _l0_body,
      grid=(_N // _BM,),
      in_specs=[
          pl.BlockSpec((_BM, _IN), lambda i: (i, 0)),
          pl.BlockSpec((_NC, _BM, _F), lambda i: (0, i, 0)),
          pl.BlockSpec((_NC, _BM, 1), lambda i: (0, i, 0)),
          pl.BlockSpec((_IN, _HID), lambda i: (0, 0)),
          pl.BlockSpec((_IN, _HID), lambda i: (0, 0)),
          pl.BlockSpec((1, _HID), lambda i: (0, 0)),
      ],
      out_specs=pl.BlockSpec((_BM, _HID), lambda i: (i, 0)),
      out_shape=jax.ShapeDtypeStruct((_N, _HID), jnp.float32),
  )(x, agg0, deg, ws, wn, b)


def _l1a(h, ws, wn, b):
  return pl.pallas_call(
      _l1a_body,
      grid=(_N // _BM,),
      in_specs=[
          pl.BlockSpec((_BM, _HID), lambda i: (i, 0)),
          pl.BlockSpec((_HID, _OUT), lambda i: (0, 0)),
          pl.BlockSpec((_HID, _OUT), lambda i: (0, 0)),
          pl.BlockSpec((1, _OUT), lambda i: (0, 0)),
      ],
      out_specs=[
          pl.BlockSpec((_BM, _OUT), lambda i: (i, 0)),
          pl.BlockSpec((_BM, _NC, _F), lambda i: (i, 0, 0)),
      ],
      out_shape=[
          jax.ShapeDtypeStruct((_N, _OUT), jnp.float32),
          jax.ShapeDtypeStruct((_N, _NC, _F), jnp.float32),
      ],
  )(h, ws, wn, b)


def _l1b(part, agg1, deg):
  return pl.pallas_call(
      _l1b_body,
      grid=(_N // _BM,),
      in_specs=[
          pl.BlockSpec((_BM, _OUT), lambda i: (i, 0)),
          pl.BlockSpec((_NC, _BM, _F), lambda i: (0, i, 0)),
          pl.BlockSpec((_NC, _BM, 1), lambda i: (0, i, 0)),
      ],
      out_specs=pl.BlockSpec((_BM, _OUT), lambda i: (i, 0)),
      out_shape=jax.ShapeDtypeStruct((_N, _OUT), jnp.float32),
  )(part, agg1, deg)


def kernel(x, edge_index, W_self0, W_neigh0, b0, W_self1, W_neigh1, b1):
  ei = edge_index.astype(jnp.int32)
  npad = _EPAD - _E
  src = jnp.concatenate([ei[0], jnp.zeros((npad,), jnp.int32)])
  dst = jnp.concatenate([ei[1], jnp.full((npad,), _NP - 1, jnp.int32)])
  srcs2 = jnp.stack([2 * src, 2 * src + 1])
  agg0, deg1 = _sc_agg_deg(x.reshape(2 * _N, _F), srcs2, dst)
  deg = deg1.reshape(_NC, _NP, 1)
  h = _l0(x, agg0, deg, W_self0, W_neigh0, b0.reshape(1, _HID))
  part, p2 = _l1a(h, W_self1, W_neigh1, b1.reshape(1, _OUT))
  (agg1,) = _sc_agg(p2.reshape(2 * _N, _F), srcs2, dst)
  out = _l1b(part, agg1, deg)
  return out


# speedup vs baseline: 1.4117x; 1.4117x over previous
import jax
import jax.numpy as jnp
from jax import lax
from jax.experimental import pallas as pl
from jax.experimental.pallas import tpu as pltpu
from jax.experimental.pallas import tpu_sc as plsc

_N = 10000
_E = 160000
_IN = 256
_HID = 512
_OUT = 256
_F = 128
_NC, _NS = 2, 16
_CH = 128
_NP = 10240
_RPS = _NP // _NS
_ZR = 128
_EPS = 10240
_NJ = _EPS // _CH
_NH = 2
_HNJ = _NJ // _NH
_EPAD = _EPS * _NS
_NCHUNK = _EPAD // _CH
_NB = 2
_BM = 400


def _make_sc_agg(with_deg):
  mesh = plsc.VectorSubcoreMesh(core_axis_name="c", subcore_axis_name="s")
  out_type = [jax.ShapeDtypeStruct((_NC, _NP, _F), jnp.float32)]
  scratch = [
      pltpu.VMEM_SHARED((_NP, _F), jnp.float32),
      pltpu.VMEM((_ZR, _F), jnp.float32),
      pltpu.VMEM((_CH,), jnp.int32),
      pltpu.VMEM((_CH,), jnp.int32),
      pltpu.VMEM((_CH, _F), jnp.float32),
      pltpu.SemaphoreType.DMA,
  ]
  if with_deg:
    out_type.append(jax.ShapeDtypeStruct((_NC, _NP), jnp.float32))
    scratch += [
        pltpu.VMEM_SHARED((_NP,), jnp.float32),
        pltpu.VMEM((_RPS,), jnp.float32),
        pltpu.VMEM((_CH,), jnp.float32),
    ]

  def body(rows2n, srcs2, dst1, *rest):
    if with_deg:
      out, deg_out = rest[0], rest[1]
      acc_sh, zbuf, idxs_v, idxd_v, rows_v, sem, deg_sh, zdeg, ones_v = rest[2:]
    else:
      out = rest[0]
      acc_sh, zbuf, idxs_v, idxd_v, rows_v, sem = rest[1:]
    c = lax.axis_index("c")
    s = lax.axis_index("s")
    zero16 = jnp.zeros((16,), jnp.float32)

    def zb(i, carry):
      for j in range(_F // 16):
        zbuf[i, pl.ds(j * 16, 16)] = zero16
      return carry
    lax.fori_loop(0, _ZR, zb, 0)

    r0 = s * _RPS
    for k in range(_RPS // _ZR):
      pltpu.sync_copy(zbuf, acc_sh.at[pl.ds(r0 + k * _ZR, _ZR)])

    if with_deg:
      def zd(i, carry):
        zdeg[pl.ds(i * 16, 16)] = zero16
        return carry
      lax.fori_loop(0, _RPS // 16, zd, 0)
      pltpu.sync_copy(zdeg, deg_sh.at[pl.ds(r0, _RPS)])
      one16 = jnp.full((16,), 1.0, jnp.float32)
      for j in range(_CH // 16):
        ones_v[pl.ds(j * 16, 16)] = one16

    plsc.subcore_barrier()

    def chunk(i, carry):
      k = s + i * _NS
      e0 = k * _CH
      pltpu.sync_copy(srcs2.at[c, pl.ds(e0, _CH)], idxs_v)
      pltpu.sync_copy(dst1.at[pl.ds(e0, _CH)], idxd_v)
      pltpu.async_copy(rows2n.at[idxs_v], rows_v, sem).wait()
      pltpu.sync_copy(rows_v, acc_sh.at[idxd_v], add=True)
      if with_deg:
        @pl.when((k < _NCHUNK // 2) == (c == 0))
        def _deg():
          pltpu.sync_copy(ones_v, deg_sh.at[idxd_v], add=True)
      return carry
    lax.fori_loop(0, _NJ, chunk, 0)

    plsc.subcore_barrier()

    for k in range(_RPS // _ZR):
      rr = r0 + k * _ZR
      pltpu.sync_copy(acc_sh.at[pl.ds(rr, _ZR)], zbuf)
      pltpu.sync_copy(zbuf, out.at[c, pl.ds(rr, _ZR)])
    if with_deg:
      pltpu.sync_copy(deg_sh.at[pl.ds(r0, _RPS)], zdeg)
      pltpu.sync_copy(zdeg, deg_out.at[c, pl.ds(r0, _RPS)])

  return pl.kernel(body, out_type=tuple(out_type), mesh=mesh,
                   scratch_types=tuple(scratch))


_sc_agg_deg = _make_sc_agg(True)
_sc_agg = _make_sc_agg(False)


def _l0_body(x_ref, agg_ref, deg_ref, ws_ref, wn_ref, b_ref, o_ref):
  inv = 1.0 / jnp.maximum(deg_ref[0] + deg_ref[1], 1.0)
  acc = jnp.dot(x_ref[...], ws_ref[...], preferred_element_type=jnp.float32)
  acc += jnp.dot(agg_ref[0] * inv, wn_ref[0:_F, :],
                 preferred_element_type=jnp.float32)
  acc += jnp.dot(agg_ref[1] * inv, wn_ref[_F:2 * _F, :],
                 preferred_element_type=jnp.float32)
  o_ref[...] = jnp.maximum(acc + b_ref[...], 0.0)


def _l1a_body(h_ref, ws_ref, wn_ref, b_ref, part_ref, p_ref):
  h = h_ref[...]
  part_ref[...] = (
      jnp.dot(h, ws_ref[...], preferred_element_type=jnp.float32) + b_ref[...])
  p = jnp.dot(h, wn_ref[...], preferred_element_type=jnp.float32)
  p_ref[:, 0, :] = p[:, 0:_F]
  p_ref[:, 1, :] = p[:, _F:2 * _F]


def _l1b_body(part_ref, agg_ref, deg_ref, o_ref):
  inv = 1.0 / jnp.maximum(deg_ref[0] + deg_ref[1], 1.0)
  o_ref[:, 0:_F] = part_ref[:, 0:_F] + agg_ref[0] * inv
  o_ref[:, _F:2 * _F] = part_ref[:, _F:2 * _F] + agg_ref[1] * inv


def _l0(x, agg0, deg, ws, wn, b):
  return pl.pallas_call(
      _l0_body,
      grid=(_N // _BM,),
      in_specs=[
          pl.BlockSpec((_BM, _IN), lambda i: (i, 0)),
          pl.BlockSpec((_NC, _BM, _F), lambda i: (0, i, 0)),
          pl.BlockSpec((_NC, _BM, 1), lambda i: (0, i, 0)),
          pl.BlockSpec((_IN, _HID), lambda i: (0, 0)),
          pl.BlockSpec((_IN, _HID), lambda i: (0, 0)),
          pl.BlockSpec((1, _HID), lambda i: (0, 0)),
      ],
      out_specs=pl.BlockSpec((_BM, _HID), lambda i: (i, 0)),
      out_shape=jax.ShapeDtypeStruct((_N, _HID), jnp.float32),
  )(x, agg0, deg, ws, wn, b)


def _l1a(h, ws, wn, b):
  return pl.pallas_call(
      _l1a_body,
      grid=(_N // _BM,),
      in_specs=[
          pl.BlockSpec((_BM, _HID), lambda i: (i, 0)),
          pl.BlockSpec((_HID, _OUT), lambda i: (0, 0)),
          pl.BlockSpec((_HID, _OUT), lambda i: (0, 0)),
          pl.BlockSpec((1, _OUT), lambda i: (0, 0)),
      ],
      out_specs=[
          pl.BlockSpec((_BM, _OUT), lambda i: (i, 0)),
          pl.BlockSpec((_BM, _NC, _F), lambda i: (i, 0, 0)),
      ],
      out_shape=[
          jax.ShapeDtypeStruct((_N, _OUT), jnp.float32),
          jax.ShapeDtypeStruct((_N, _NC, _F), jnp.float32),
      ],
  )(h, ws, wn, b)


def _l1b(part, agg1, deg):
  return pl.pallas_call(
      _l1b_body,
      grid=(_N // _BM,),
      in_specs=[
          pl.BlockSpec((_BM, _OUT), lambda i: (i, 0)),
          pl.BlockSpec((_NC, _BM, _F), lambda i: (0, i, 0)),
          pl.BlockSpec((_NC, _BM, 1), lambda i: (0, i, 0)),
      ],
      out_specs=pl.BlockSpec((_BM, _OUT), lambda i: (i, 0)),
      out_shape=jax.ShapeDtypeStruct((_N, _OUT), jnp.float32),
  )(part, agg1, deg)


def kernel(x, edge_index, W_self0, W_neigh0, b0, W_self1, W_neigh1, b1):
  ei = edge_index.astype(jnp.int32)
  npad = _EPAD - _E
  filler = jnp.arange(npad, dtype=jnp.int32)
  src = jnp.concatenate([ei[0], filler % _N])
  dst = jnp.concatenate([ei[1], _N + (filler % (_NP - _N))])
  srcs2 = jnp.stack([2 * src, 2 * src + 1])
  agg0, deg1 = _sc_agg_deg(x.reshape(2 * _N, _F), srcs2, dst)
  deg = deg1.reshape(_NC, _NP, 1)
  h = _l0(x, agg0, deg, W_self0, W_neigh0, b0.reshape(1, _HID))
  part, p2 = _l1a(h, W_self1, W_neigh1, b1.reshape(1, _OUT))
  (agg1,) = _sc_agg(p2.reshape(2 * _N, _F), srcs2, dst)
  out = _l1b(part, agg1, deg)
  return out


# speedup vs baseline: 1.8322x; 1.2979x over previous
import jax
import jax.numpy as jnp
from jax import lax
from jax.experimental import pallas as pl
from jax.experimental.pallas import tpu as pltpu
from jax.experimental.pallas import tpu_sc as plsc

_N = 10000
_E = 160000
_IN = 256
_HID = 512
_OUT = 256
_F = 128
_NC, _NS = 2, 16
_CH = 128
_NP = 10240
_RPS = _NP // _NS
_ZR = 128
_EPS = 10240
_NJ = _EPS // _CH
_NH = 2
_HNJ = _NJ // _NH
_EPAD = _EPS * _NS
_NCHUNK = _EPAD // _CH
_NB = 2
_BM = 400


def _make_sc_agg(with_deg):
  mesh = plsc.VectorSubcoreMesh(core_axis_name="c", subcore_axis_name="s")
  out_type = [jax.ShapeDtypeStruct((_NC, _NP, _F), jnp.float32)]
  scratch = [
      pltpu.VMEM_SHARED((_NP, _F), jnp.float32),
      pltpu.VMEM((_ZR, _F), jnp.float32),
      pltpu.VMEM((_CH,), jnp.int32),
      pltpu.VMEM((_CH,), jnp.int32),
      pltpu.VMEM((_CH,), jnp.int32),
      pltpu.VMEM((_CH,), jnp.int32),
      pltpu.VMEM((_CH, _F), jnp.float32),
      pltpu.SemaphoreType.DMA,
      pltpu.SemaphoreType.DMA,
      pltpu.SemaphoreType.DMA,
  ]
  if with_deg:
    out_type.append(jax.ShapeDtypeStruct((_NC, _NP), jnp.float32))
    scratch += [
        pltpu.VMEM_SHARED((_NP,), jnp.float32),
        pltpu.VMEM((_RPS,), jnp.float32),
        pltpu.VMEM((_CH,), jnp.float32),
    ]

  def body(rows2n, srcs2, dst1, *rest):
    if with_deg:
      out, deg_out = rest[0], rest[1]
      (acc_sh, zbuf, idxs0, idxd0, idxs1, idxd1, rows_v, sem, semi0, semi1,
       deg_sh, zdeg, ones_v) = rest[2:]
    else:
      out = rest[0]
      (acc_sh, zbuf, idxs0, idxd0, idxs1, idxd1, rows_v, sem,
       semi0, semi1) = rest[1:]
    idxs = (idxs0, idxs1)
    idxd = (idxd0, idxd1)
    semi = (semi0, semi1)
    c = lax.axis_index("c")
    s = lax.axis_index("s")
    zero16 = jnp.zeros((16,), jnp.float32)

    def zb(i, carry):
      for j in range(_F // 16):
        zbuf[i, pl.ds(j * 16, 16)] = zero16
      return carry
    lax.fori_loop(0, _ZR, zb, 0)

    r0 = s * _RPS
    for k in range(_RPS // _ZR):
      pltpu.sync_copy(zbuf, acc_sh.at[pl.ds(r0 + k * _ZR, _ZR)])

    if with_deg:
      def zd(i, carry):
        zdeg[pl.ds(i * 16, 16)] = zero16
        return carry
      lax.fori_loop(0, _RPS // 16, zd, 0)
      pltpu.sync_copy(zdeg, deg_sh.at[pl.ds(r0, _RPS)])
      one16 = jnp.full((16,), 1.0, jnp.float32)
      for j in range(_CH // 16):
        ones_v[pl.ds(j * 16, 16)] = one16

    plsc.subcore_barrier()

    def idx_start(i, q):
      e0 = (s + i * _NS) * _CH
      pltpu.async_copy(srcs2.at[c, pl.ds(e0, _CH)], idxs[q], semi[q])
      pltpu.async_copy(dst1.at[pl.ds(e0, _CH)], idxd[q], semi[q])

    def idx_wait(i, q):
      e0 = (s + i * _NS) * _CH
      pltpu.make_async_copy(srcs2.at[c, pl.ds(e0, _CH)], idxs[q],
                            semi[q]).wait()
      pltpu.make_async_copy(dst1.at[pl.ds(e0, _CH)], idxd[q],
                            semi[q]).wait()

    idx_start(0, 0)

    def chunk2(ii, carry):
      for q in range(2):
        i = 2 * ii + q
        k = s + i * _NS
        idx_wait(i, q)

        @pl.when(i + 1 < _NJ)
        def _pf():
          idx_start(i + 1, 1 - q)
        pltpu.async_copy(rows2n.at[idxs[q]], rows_v, sem).wait()
        pltpu.sync_copy(rows_v, acc_sh.at[idxd[q]], add=True)
        if with_deg:
          @pl.when((k < _NCHUNK // 2) == (c == 0))
          def _deg():
            pltpu.sync_copy(ones_v, deg_sh.at[idxd[q]], add=True)
      return carry
    lax.fori_loop(0, _NJ // 2, chunk2, 0)

    plsc.subcore_barrier()

    for k in range(_RPS // _ZR):
      rr = r0 + k * _ZR
      pltpu.sync_copy(acc_sh.at[pl.ds(rr, _ZR)], zbuf)
      pltpu.sync_copy(zbuf, out.at[c, pl.ds(rr, _ZR)])
    if with_deg:
      pltpu.sync_copy(deg_sh.at[pl.ds(r0, _RPS)], zdeg)
      pltpu.sync_copy(zdeg, deg_out.at[c, pl.ds(r0, _RPS)])

  return pl.kernel(body, out_type=tuple(out_type), mesh=mesh,
                   scratch_types=tuple(scratch))


_sc_agg_deg = _make_sc_agg(True)
_sc_agg = _make_sc_agg(False)


def _l0_body(x_ref, agg_ref, deg_ref, ws_ref, wn_ref, b_ref, o_ref):
  inv = 1.0 / jnp.maximum(deg_ref[0] + deg_ref[1], 1.0)
  acc = jnp.dot(x_ref[...], ws_ref[...], preferred_element_type=jnp.float32)
  acc += jnp.dot(agg_ref[0] * inv, wn_ref[0:_F, :],
                 preferred_element_type=jnp.float32)
  acc += jnp.dot(agg_ref[1] * inv, wn_ref[_F:2 * _F, :],
                 preferred_element_type=jnp.float32)
  o_ref[...] = jnp.maximum(acc + b_ref[...], 0.0)


def _l1a_body(h_ref, ws_ref, wn_ref, b_ref, part_ref, p_ref):
  h = h_ref[...]
  part_ref[...] = (
      jnp.dot(h, ws_ref[...], preferred_element_type=jnp.float32) + b_ref[...])
  p = jnp.dot(h, wn_ref[...], preferred_element_type=jnp.float32)
  p_ref[:, 0, :] = p[:, 0:_F]
  p_ref[:, 1, :] = p[:, _F:2 * _F]


def _l1b_body(part_ref, agg_ref, deg_ref, o_ref):
  inv = 1.0 / jnp.maximum(deg_ref[0] + deg_ref[1], 1.0)
  o_ref[:, 0:_F] = part_ref[:, 0:_F] + agg_ref[0] * inv
  o_ref[:, _F:2 * _F] = part_ref[:, _F:2 * _F] + agg_ref[1] * inv


def _l0(x, agg0, deg, ws, wn, b):
  return pl.pallas_call(
      _l0_body,
      grid=(_N // _BM,),
      in_specs=[
          pl.BlockSpec((_BM, _IN), lambda i: (i, 0)),
          pl.BlockSpec((_NC, _BM, _F), lambda i: (0, i, 0)),
          pl.BlockSpec((_NC, _BM, 1), lambda i: (0, i, 0)),
          pl.BlockSpec((_IN, _HID), lambda i: (0, 0)),
          pl.BlockSpec((_IN, _HID), lambda i: (0, 0)),
          pl.BlockSpec((1, _HID), lambda i: (0, 0)),
      ],
      out_specs=pl.BlockSpec((_BM, _HID), lambda i: (i, 0)),
      out_shape=jax.ShapeDtypeStruct((_N, _HID), jnp.float32),
  )(x, agg0, deg, ws, wn, b)


def _l1a(h, ws, wn, b):
  return pl.pallas_call(
      _l1a_body,
      grid=(_N // _BM,),
      in_specs=[
          pl.BlockSpec((_BM, _HID), lambda i: (i, 0)),
          pl.BlockSpec((_HID, _OUT), lambda i: (0, 0)),
          pl.BlockSpec((_HID, _OUT), lambda i: (0, 0)),
          pl.BlockSpec((1, _OUT), lambda i: (0, 0)),
      ],
      out_specs=[
          pl.BlockSpec((_BM, _OUT), lambda i: (i, 0)),
          pl.BlockSpec((_BM, _NC, _F), lambda i: (i, 0, 0)),
      ],
      out_shape=[
          jax.ShapeDtypeStruct((_N, _OUT), jnp.float32),
          jax.ShapeDtypeStruct((_N, _NC, _F), jnp.float32),
      ],
  )(h, ws, wn, b)


def _l1b(part, agg1, deg):
  return pl.pallas_call(
      _l1b_body,
      grid=(_N // _BM,),
      in_specs=[
          pl.BlockSpec((_BM, _OUT), lambda i: (i, 0)),
          pl.BlockSpec((_NC, _BM, _F), lambda i: (0, i, 0)),
          pl.BlockSpec((_NC, _BM, 1), lambda i: (0, i, 0)),
      ],
      out_specs=pl.BlockSpec((_BM, _OUT), lambda i: (i, 0)),
      out_shape=jax.ShapeDtypeStruct((_N, _OUT), jnp.float32),
  )(part, agg1, deg)


def kernel(x, edge_index, W_self0, W_neigh0, b0, W_self1, W_neigh1, b1):
  ei = edge_index.astype(jnp.int32)
  npad = _EPAD - _E
  filler = jnp.arange(npad, dtype=jnp.int32)
  src = jnp.concatenate([ei[0], filler % _N])
  dst = jnp.concatenate([ei[1], _N + (filler % (_NP - _N))])
  srcs2 = jnp.stack([2 * src, 2 * src + 1])
  agg0, deg1 = _sc_agg_deg(x.reshape(2 * _N, _F), srcs2, dst)
  deg = deg1.reshape(_NC, _NP, 1)
  h = _l0(x, agg0, deg, W_self0, W_neigh0, b0.reshape(1, _HID))
  part, p2 = _l1a(h, W_self1, W_neigh1, b1.reshape(1, _OUT))
  (agg1,) = _sc_agg(p2.reshape(2 * _N, _F), srcs2, dst)
  out = _l1b(part, agg1, deg)
  return out


# speedup vs baseline: 2.6032x; 1.4208x over previous
import jax
import jax.numpy as jnp
from jax import lax
from jax.experimental import pallas as pl
from jax.experimental.pallas import tpu as pltpu
from jax.experimental.pallas import tpu_sc as plsc

_N = 10000
_E = 160000
_IN = 256
_HID = 512
_OUT = 256
_F = 128
_NC, _NS = 2, 16
_CH = 128
_NP = 10240
_RPS = _NP // _NS
_ZR = 128
_EPS = 10240
_NJ = _EPS // _CH
_NH = 2
_HNJ = _NJ // _NH
_EPAD = _EPS * _NS
_NCHUNK = _EPAD // _CH
_NB = 2
_BM = 400


def _make_sc_agg(with_deg):
  mesh = plsc.VectorSubcoreMesh(core_axis_name="c", subcore_axis_name="s")
  out_type = [jax.ShapeDtypeStruct((_NC, _NP, _F), jnp.float32)]
  scratch = [
      pltpu.VMEM_SHARED((_NP, _F), jnp.float32),
  ] + [pltpu.VMEM((_CH,), jnp.int32) for _ in range(8)] \
    + [pltpu.VMEM((_CH, _F), jnp.float32) for _ in range(2)] \
    + [pltpu.SemaphoreType.DMA for _ in range(8)]
  if with_deg:
    out_type.append(jax.ShapeDtypeStruct((_NC, _NP), jnp.float32))
    scratch += [
        pltpu.VMEM_SHARED((_NP,), jnp.float32),
        pltpu.VMEM((_RPS,), jnp.float32),
        pltpu.VMEM((_CH,), jnp.float32),
        pltpu.SemaphoreType.DMA,
        pltpu.SemaphoreType.DMA,
    ]

  def body(rows2n, srcs2, dst1, *rest):
    if with_deg:
      out, deg_out = rest[0], rest[1]
      deg_sh, zdeg, ones_v, semd0, semd1 = rest[21:]
      rest = rest[2:21]
    else:
      out = rest[0]
      rest = rest[1:20]
    acc_sh = rest[0]
    idxs = rest[1:9:2]
    idxd = rest[2:9:2]
    rows = rest[9:11]
    semi = rest[11:15]
    sem_g = rest[15:17]
    sem_s = rest[17:19]
    if with_deg:
      sem_d = (semd0, semd1)
    c = lax.axis_index("c")
    s = lax.axis_index("s")
    zero16 = jnp.zeros((16,), jnp.float32)
    zbuf = rows[0]

    def zb(i, carry):
      for j in range(_F // 16):
        zbuf[i, pl.ds(j * 16, 16)] = zero16
      return carry
    lax.fori_loop(0, _ZR, zb, 0)

    r0 = s * _RPS
    for k in range(_RPS // _ZR):
      pltpu.sync_copy(zbuf, acc_sh.at[pl.ds(r0 + k * _ZR, _ZR)])

    if with_deg:
      def zd(i, carry):
        zdeg[pl.ds(i * 16, 16)] = zero16
        return carry
      lax.fori_loop(0, _RPS // 16, zd, 0)
      pltpu.sync_copy(zdeg, deg_sh.at[pl.ds(r0, _RPS)])
      one16 = jnp.full((16,), 1.0, jnp.float32)
      for j in range(_CH // 16):
        ones_v[pl.ds(j * 16, 16)] = one16

    plsc.subcore_barrier()

    def idx_start(i, p):
      e0 = (s + i * _NS) * _CH
      pltpu.async_copy(srcs2.at[c, pl.ds(e0, _CH)], idxs[p], semi[p])
      pltpu.async_copy(dst1.at[pl.ds(e0, _CH)], idxd[p], semi[p])

    def idx_wait(i, p):
      e0 = (s + i * _NS) * _CH
      pltpu.make_async_copy(srcs2.at[c, pl.ds(e0, _CH)], idxs[p],
                            semi[p]).wait()
      pltpu.make_async_copy(dst1.at[pl.ds(e0, _CH)], idxd[p],
                            semi[p]).wait()

    def deg_cond(i):
      return ((s + i * _NS) < _NCHUNK // 2) == (c == 0)

    def scat_start(i, p, r):
      pltpu.async_copy(rows[r], acc_sh.at[idxd[p]], sem_s[r], add=True)
      if with_deg:
        @pl.when(deg_cond(i))
        def _deg():
          pltpu.async_copy(ones_v, deg_sh.at[idxd[p]], sem_d[r], add=True)

    def scat_wait(i, p, r):
      pltpu.make_async_copy(rows[r], acc_sh.at[idxd[p]], sem_s[r]).wait()
      if with_deg:
        @pl.when(deg_cond(i))
        def _dw():
          pltpu.make_async_copy(ones_v, deg_sh.at[idxd[p]], sem_d[r]).wait()

    idx_start(0, 0)

    def chunk4(ii, carry):
      for p in range(4):
        i = 4 * ii + p
        r = p % 2
        idx_wait(i, p)

        @pl.when(i + 1 < _NJ)
        def _pf():
          idx_start(i + 1, (p + 1) % 4)

        @pl.when(i >= 2)
        def _sw():
          scat_wait(i - 2, (p + 2) % 4, r)
        pltpu.async_copy(rows2n.at[idxs[p]], rows[r], sem_g[r])

        @pl.when(i >= 1)
        def _sc():
          pltpu.make_async_copy(rows2n.at[idxs[(p + 3) % 4]], rows[1 - r],
                                sem_g[1 - r]).wait()
          scat_start(i - 1, (p + 3) % 4, 1 - r)
      return carry
    lax.fori_loop(0, _NJ // 4, chunk4, 0)

    m = _NJ - 1
    pltpu.make_async_copy(rows2n.at[idxs[3]], rows[1], sem_g[1]).wait()
    scat_start(m, 3, 1)
    scat_wait(m - 1, 2, 0)
    scat_wait(m, 3, 1)

    plsc.subcore_barrier()

    for k in range(_RPS // _ZR):
      rr = r0 + k * _ZR
      pltpu.sync_copy(acc_sh.at[pl.ds(rr, _ZR)], zbuf)
      pltpu.sync_copy(zbuf, out.at[c, pl.ds(rr, _ZR)])
    if with_deg:
      pltpu.sync_copy(deg_sh.at[pl.ds(r0, _RPS)], zdeg)
      pltpu.sync_copy(zdeg, deg_out.at[c, pl.ds(r0, _RPS)])

  return pl.kernel(body, out_type=tuple(out_type), mesh=mesh,
                   scratch_types=tuple(scratch))


_sc_agg_deg = _make_sc_agg(True)
_sc_agg = _make_sc_agg(False)


def _l0_body(x_ref, agg_ref, deg_ref, ws_ref, wn_ref, b_ref, o_ref):
  inv = 1.0 / jnp.maximum(deg_ref[0] + deg_ref[1], 1.0)
  acc = jnp.dot(x_ref[...], ws_ref[...], preferred_element_type=jnp.float32)
  acc += jnp.dot(agg_ref[0] * inv, wn_ref[0:_F, :],
                 preferred_element_type=jnp.float32)
  acc += jnp.dot(agg_ref[1] * inv, wn_ref[_F:2 * _F, :],
                 preferred_element_type=jnp.float32)
  o_ref[...] = jnp.maximum(acc + b_ref[...], 0.0)


def _l1a_body(h_ref, ws_ref, wn_ref, b_ref, part_ref, p_ref):
  h = h_ref[...]
  part_ref[...] = (
      jnp.dot(h, ws_ref[...], preferred_element_type=jnp.float32) + b_ref[...])
  p = jnp.dot(h, wn_ref[...], preferred_element_type=jnp.float32)
  p_ref[:, 0, :] = p[:, 0:_F]
  p_ref[:, 1, :] = p[:, _F:2 * _F]


def _l1b_body(part_ref, agg_ref, deg_ref, o_ref):
  inv = 1.0 / jnp.maximum(deg_ref[0] + deg_ref[1], 1.0)
  o_ref[:, 0:_F] = part_ref[:, 0:_F] + agg_ref[0] * inv
  o_ref[:, _F:2 * _F] = part_ref[:, _F:2 * _F] + agg_ref[1] * inv


def _l0(x, agg0, deg, ws, wn, b):
  return pl.pallas_call(
      _l0_body,
      grid=(_N // _BM,),
      in_specs=[
          pl.BlockSpec((_BM, _IN), lambda i: (i, 0)),
          pl.BlockSpec((_NC, _BM, _F), lambda i: (0, i, 0)),
          pl.BlockSpec((_NC, _BM, 1), lambda i: (0, i, 0)),
          pl.BlockSpec((_IN, _HID), lambda i: (0, 0)),
          pl.BlockSpec((_IN, _HID), lambda i: (0, 0)),
          pl.BlockSpec((1, _HID), lambda i: (0, 0)),
      ],
      out_specs=pl.BlockSpec((_BM, _HID), lambda i: (i, 0)),
      out_shape=jax.ShapeDtypeStruct((_N, _HID), jnp.float32),
  )(x, agg0, deg, ws, wn, b)


def _l1a(h, ws, wn, b):
  return pl.pallas_call(
      _l1a_body,
      grid=(_N // _BM,),
      in_specs=[
          pl.BlockSpec((_BM, _HID), lambda i: (i, 0)),
          pl.BlockSpec((_HID, _OUT), lambda i: (0, 0)),
          pl.BlockSpec((_HID, _OUT), lambda i: (0, 0)),
          pl.BlockSpec((1, _OUT), lambda i: (0, 0)),
      ],
      out_specs=[
          pl.BlockSpec((_BM, _OUT), lambda i: (i, 0)),
          pl.BlockSpec((_BM, _NC, _F), lambda i: (i, 0, 0)),
      ],
      out_shape=[
          jax.ShapeDtypeStruct((_N, _OUT), jnp.float32),
          jax.ShapeDtypeStruct((_N, _NC, _F), jnp.float32),
      ],
  )(h, ws, wn, b)


def _l1b(part, agg1, deg):
  return pl.pallas_call(
      _l1b_body,
      grid=(_N // _BM,),
      in_specs=[
          pl.BlockSpec((_BM, _OUT), lambda i: (i, 0)),
          pl.BlockSpec((_NC, _BM, _F), lambda i: (0, i, 0)),
          pl.BlockSpec((_NC, _BM, 1), lambda i: (0, i, 0)),
      ],
      out_specs=pl.BlockSpec((_BM, _OUT), lambda i: (i, 0)),
      out_shape=jax.ShapeDtypeStruct((_N, _OUT), jnp.float32),
  )(part, agg1, deg)


def kernel(x, edge_index, W_self0, W_neigh0, b0, W_self1, W_neigh1, b1):
  ei = edge_index.astype(jnp.int32)
  npad = _EPAD - _E
  filler = jnp.arange(npad, dtype=jnp.int32)
  src = jnp.concatenate([ei[0], filler % _N])
  dst = jnp.concatenate([ei[1], _N + (filler % (_NP - _N))])
  srcs2 = jnp.stack([2 * src, 2 * src + 1])
  agg0, deg1 = _sc_agg_deg(x.reshape(2 * _N, _F), srcs2, dst)
  deg = deg1.reshape(_NC, _NP, 1)
  h = _l0(x, agg0, deg, W_self0, W_neigh0, b0.reshape(1, _HID))
  part, p2 = _l1a(h, W_self1, W_neigh1, b1.reshape(1, _OUT))
  (agg1,) = _sc_agg(p2.reshape(2 * _N, _F), srcs2, dst)
  out = _l1b(part, agg1, deg)
  return out
